# baseline (speedup 1.0000x reference)
import functools

import jax
import jax.numpy as jnp
from jax import lax
from jax.experimental import pallas as pl
from jax.experimental.pallas import tpu as pltpu
from jax.experimental.pallas import tpu_sc as plsc

DIM = 64
BATCH = 16384
NC = 2
NS = 16
NW = NC * NS
BPW = BATCH // NW
ROUND = 64
NROUND = BPW // ROUND
GPR = ROUND // 16


def _dist_mult_body(h_hbm, r_hbm, t_hbm, ent3, rel2, out_hbm,
                    hidx, ridx, tidx, relv, hbuf, tbuf, outv,
                    sem0, sem1, semr):
    wid = lax.axis_index("s") * NC + lax.axis_index("c")
    base = wid * BPW

    pltpu.sync_copy(h_hbm.at[pl.ds(wid, 1)], hidx)
    pltpu.sync_copy(r_hbm.at[pl.ds(wid, 1)], ridx)
    pltpu.sync_copy(t_hbm.at[pl.ds(wid, 1)], tidx)
    rel_cp = pltpu.async_copy(rel2, relv, semr)

    sems = (sem0, sem1)

    def issue_round(c, slot):
        sem = sems[c % 2]
        hb, tb = hbuf.at[slot], tbuf.at[slot]

        def issue(j, _):
            off = c * ROUND + j * 16
            hv = hidx[0, pl.ds(off, 16)]
            tv = tidx[0, pl.ds(off, 16)]
            for k in range(16):
                row = j * 16 + k
                hs, ts = hv[k], tv[k]
                pltpu.async_copy(ent3.at[hs >> 3, hs & 7],
                                 hb.at[row >> 3, row & 7], sem)
                pltpu.async_copy(ent3.at[ts >> 3, ts & 7],
                                 tb.at[row >> 3, row & 7], sem)
            return 0

        lax.fori_loop(0, GPR, issue, 0)

    def drain_round(c, slot):
        sem = sems[c % 2]
        dummy = ent3.at[pl.ds(0, ROUND // 8)]
        pltpu.make_async_copy(dummy, hbuf.at[slot], sem).wait()
        pltpu.make_async_copy(dummy, tbuf.at[slot], sem).wait()

    iot = lax.iota(jnp.int32, 16)
    dnums = lax.GatherDimensionNumbers(
        offset_dims=(), collapsed_slice_dims=(0,), start_index_map=(0,))

    def lane_gather(v, idx):
        return lax.gather(v, idx[:, None], dnums, slice_sizes=(1,),
                          mode=lax.GatherScatterMode.PROMISE_IN_BOUNDS)

    def hsum_splat(v):
        for sh in (8, 4, 2, 1):
            idx = (iot + sh) & 15
            v = v + lane_gather(v, idx)
        return v

    def compute_round(c, slot):
        hb, tb = hbuf.at[slot], tbuf.at[slot]

        def group(g, carry):
            off = c * ROUND + g * 16
            rv = ridx[0, pl.ds(off, 16)]
            vec = jnp.zeros((16,), jnp.float32)
            for k16 in range(16):
                row = g * 16 + k16
                rs = rv[k16]
                rrow = rs >> 1
                roff = (rs & 1) << 6
                p = jnp.zeros((16,), jnp.float32)
                for k in range(DIM // 16):
                    sl = pl.ds(k * 16, 16)
                    p = p + (hb[row >> 3, row & 7, sl]
                             * relv[rrow, pl.ds(roff + k * 16, 16)]
                             * tb[row >> 3, row & 7, sl])
                s = hsum_splat(p)
                vec = jnp.where(iot == k16, s, vec)
            outv[pl.ds(pl.multiple_of(c * ROUND + g * 16, 16), 16)] = vec
            return carry

        lax.fori_loop(0, GPR, group, 0)

    issue_round(0, 0)
    for c in range(NROUND):
        if c + 1 < NROUND:
            issue_round(c + 1, (c + 1) % 2)
        drain_round(c, c % 2)
        if c == 0:
            rel_cp.wait()
        compute_round(c, c % 2)

    pltpu.sync_copy(outv, out_hbm.at[pl.ds(base, BPW)])


@jax.jit
def _dist_mult_sc(h2, r2, t2, ent3, rel2):
    mesh = plsc.VectorSubcoreMesh(core_axis_name="c", subcore_axis_name="s")
    kfn = functools.partial(
        pl.kernel,
        out_type=jax.ShapeDtypeStruct((BATCH,), jnp.float32),
        mesh=mesh,
        scratch_types=[
            pltpu.VMEM((1, BPW), jnp.int32),
            pltpu.VMEM((1, BPW), jnp.int32),
            pltpu.VMEM((1, BPW), jnp.int32),
            pltpu.VMEM((500, 128), jnp.float32),
            pltpu.VMEM((2, ROUND // 8, 8, DIM), jnp.float32),
            pltpu.VMEM((2, ROUND // 8, 8, DIM), jnp.float32),
            pltpu.VMEM((BPW,), jnp.float32),
            pltpu.SemaphoreType.DMA,
            pltpu.SemaphoreType.DMA,
            pltpu.SemaphoreType.DMA,
        ],
    )(_dist_mult_body)
    return kfn(h2, r2, t2, ent3, rel2)


def kernel(h, r, t, ent_emb, rel_emb):
    h2 = jnp.asarray(h, jnp.int32).reshape(NW, BPW)
    r2 = jnp.asarray(r, jnp.int32).reshape(NW, BPW)
    t2 = jnp.asarray(t, jnp.int32).reshape(NW, BPW)
    ent3 = ent_emb.reshape(ent_emb.shape[0] // 8, 8, DIM)
    rel2 = rel_emb.reshape(rel_emb.shape[0] // 2, 128)
    return _dist_mult_sc(h2, r2, t2, ent3, rel2)


# speedup vs baseline: 1.0268x; 1.0268x over previous
import functools

import jax
import jax.numpy as jnp
from jax import lax
from jax.experimental import pallas as pl
from jax.experimental.pallas import tpu as pltpu
from jax.experimental.pallas import tpu_sc as plsc

DIM = 64
BATCH = 16384
NC = 2
NS = 16
NW = NC * NS
BPW = BATCH // NW
ROUND = 128
NROUND = BPW // ROUND
GPR = ROUND // 16


def _dist_mult_body(h_hbm, r_hbm, t_hbm, ent3, rel3, out_hbm,
                    hidx, ridx, tidx, hbuf, rbuf, tbuf, outv, sem0, sem1):
    wid = lax.axis_index("s") * NC + lax.axis_index("c")
    base = wid * BPW

    pltpu.sync_copy(h_hbm.at[pl.ds(wid, 1)], hidx)
    pltpu.sync_copy(r_hbm.at[pl.ds(wid, 1)], ridx)
    pltpu.sync_copy(t_hbm.at[pl.ds(wid, 1)], tidx)

    sems = (sem0, sem1)

    def issue_round(c, slot):
        sem = sems[c % 2]
        hb, rb, tb = hbuf.at[slot], rbuf.at[slot], tbuf.at[slot]

        def issue(j, _):
            off = c * ROUND + j * 16
            hv = hidx[0, pl.ds(off, 16)]
            rv = ridx[0, pl.ds(off, 16)]
            tv = tidx[0, pl.ds(off, 16)]
            for k in range(16):
                row = j * 16 + k
                hs, rs, ts = hv[k], rv[k], tv[k]
                pltpu.async_copy(ent3.at[hs >> 3, hs & 7], hb.at[row], sem)
                pltpu.async_copy(rel3.at[rs >> 3, rs & 7], rb.at[row], sem)
                pltpu.async_copy(ent3.at[ts >> 3, ts & 7], tb.at[row], sem)
            return 0

        lax.fori_loop(0, GPR, issue, 0)

    def drain_round(c, slot):
        sem = sems[c % 2]
        dummy = ent3.at[pl.ds(0, ROUND // 8)]
        pltpu.make_async_copy(dummy, hbuf.at[slot], sem).wait()
        pltpu.make_async_copy(dummy, rbuf.at[slot], sem).wait()
        pltpu.make_async_copy(dummy, tbuf.at[slot], sem).wait()

    iot = lax.iota(jnp.int32, 16)
    dnums = lax.GatherDimensionNumbers(
        offset_dims=(), collapsed_slice_dims=(0,), start_index_map=(0,))

    def lane_gather(v, idx):
        return lax.gather(v, idx[:, None], dnums, slice_sizes=(1,),
                          mode=lax.GatherScatterMode.PROMISE_IN_BOUNDS)

    def hsum_splat(v):
        for sh in (8, 4, 2, 1):
            idx = (iot + sh) & 15
            v = v + lane_gather(v, idx)
        return v

    def compute_round(c, slot):
        hb, rb, tb = hbuf.at[slot], rbuf.at[slot], tbuf.at[slot]

        def group(g, carry):
            vec = jnp.zeros((16,), jnp.float32)
            for k16 in range(16):
                row = g * 16 + k16
                p = jnp.zeros((16,), jnp.float32)
                for k in range(DIM // 16):
                    sl = pl.ds(k * 16, 16)
                    p = p + hb[row, sl] * rb[row, sl] * tb[row, sl]
                s = hsum_splat(p)
                vec = jnp.where(iot == k16, s, vec)
            outv[pl.ds(pl.multiple_of(c * ROUND + g * 16, 16), 16)] = vec
            return carry

        lax.fori_loop(0, GPR, group, 0)

    issue_round(0, 0)
    for c in range(NROUND):
        if c + 1 < NROUND:
            issue_round(c + 1, (c + 1) % 2)
        drain_round(c, c % 2)
        compute_round(c, c % 2)

    pltpu.sync_copy(outv, out_hbm.at[pl.ds(base, BPW)])


@jax.jit
def _dist_mult_sc(h2, r2, t2, ent3, rel3):
    mesh = plsc.VectorSubcoreMesh(core_axis_name="c", subcore_axis_name="s")
    kfn = functools.partial(
        pl.kernel,
        out_type=jax.ShapeDtypeStruct((BATCH,), jnp.float32),
        mesh=mesh,
        scratch_types=[
            pltpu.VMEM((1, BPW), jnp.int32),
            pltpu.VMEM((1, BPW), jnp.int32),
            pltpu.VMEM((1, BPW), jnp.int32),
            pltpu.VMEM((2, ROUND, DIM), jnp.float32),
            pltpu.VMEM((2, ROUND, DIM), jnp.float32),
            pltpu.VMEM((2, ROUND, DIM), jnp.float32),
            pltpu.VMEM((BPW,), jnp.float32),
            pltpu.SemaphoreType.DMA,
            pltpu.SemaphoreType.DMA,
        ],
    )(_dist_mult_body)
    return kfn(h2, r2, t2, ent3, rel3)


def kernel(h, r, t, ent_emb, rel_emb):
    h2 = jnp.asarray(h, jnp.int32).reshape(NW, BPW)
    r2 = jnp.asarray(r, jnp.int32).reshape(NW, BPW)
    t2 = jnp.asarray(t, jnp.int32).reshape(NW, BPW)
    ent3 = ent_emb.reshape(ent_emb.shape[0] // 8, 8, DIM)
    rel3 = rel_emb.reshape(rel_emb.shape[0] // 8, 8, DIM)
    return _dist_mult_sc(h2, r2, t2, ent3, rel3)


# speedup vs baseline: 1.0326x; 1.0057x over previous
import functools

import jax
import jax.numpy as jnp
from jax import lax
from jax.experimental import pallas as pl
from jax.experimental.pallas import tpu as pltpu
from jax.experimental.pallas import tpu_sc as plsc

DIM = 64
BATCH = 16384
NC = 2
NS = 16
NW = NC * NS
BPW = BATCH // NW
ROUND = 128
NROUND = BPW // ROUND
GPR = ROUND // 16


def _dist_mult_body(idx_hbm, ent3, rel3, out_hbm,
                    idxv, hbuf, rbuf, tbuf, outv, sem0, sem1):
    wid = lax.axis_index("s") * NC + lax.axis_index("c")
    base = wid * BPW

    pltpu.sync_copy(idx_hbm.at[pl.ds(wid, 1)], idxv)
    hidx, ridx, tidx = idxv.at[0, 0, 0], idxv.at[0, 1, 0], idxv.at[0, 2, 0]

    sems = (sem0, sem1)

    def issue_round(c, slot):
        sem = sems[c % 2]
        hb, rb, tb = hbuf.at[slot], rbuf.at[slot], tbuf.at[slot]

        def issue(j, _):
            off = c * ROUND + j * 16
            hv = hidx[pl.ds(off, 16)]
            rv = ridx[pl.ds(off, 16)]
            tv = tidx[pl.ds(off, 16)]
            for k in range(16):
                row = j * 16 + k
                hs, rs, ts = hv[k], rv[k], tv[k]
                pltpu.async_copy(ent3.at[hs >> 3, hs & 7], hb.at[row], sem)
                pltpu.async_copy(rel3.at[rs >> 3, rs & 7], rb.at[row], sem)
                pltpu.async_copy(ent3.at[ts >> 3, ts & 7], tb.at[row], sem)
            return 0

        lax.fori_loop(0, GPR, issue, 0)

    def drain_round(c, slot):
        sem = sems[c % 2]
        dummy = ent3.at[pl.ds(0, ROUND // 8)]
        pltpu.make_async_copy(dummy, hbuf.at[slot], sem).wait()
        pltpu.make_async_copy(dummy, rbuf.at[slot], sem).wait()
        pltpu.make_async_copy(dummy, tbuf.at[slot], sem).wait()

    iot = lax.iota(jnp.int32, 16)
    dnums = lax.GatherDimensionNumbers(
        offset_dims=(), collapsed_slice_dims=(0,), start_index_map=(0,))

    def lane_gather(v, idx):
        return lax.gather(v, idx[:, None], dnums, slice_sizes=(1,),
                          mode=lax.GatherScatterMode.PROMISE_IN_BOUNDS)

    def hsum_splat(v):
        for sh in (8, 4, 2, 1):
            idx = (iot + sh) & 15
            v = v + lane_gather(v, idx)
        return v

    def compute_round(c, slot):
        hb, rb, tb = hbuf.at[slot], rbuf.at[slot], tbuf.at[slot]

        def group(g, carry):
            vec = jnp.zeros((16,), jnp.float32)
            for k16 in range(16):
                row = g * 16 + k16
                p = jnp.zeros((16,), jnp.float32)
                for k in range(DIM // 16):
                    sl = pl.ds(k * 16, 16)
                    p = p + hb[row, sl] * rb[row, sl] * tb[row, sl]
                s = hsum_splat(p)
                vec = jnp.where(iot == k16, s, vec)
            outv[pl.ds(pl.multiple_of(c * ROUND + g * 16, 16), 16)] = vec
            return carry

        lax.fori_loop(0, GPR, group, 0)

    issue_round(0, 0)
    for c in range(NROUND):
        if c + 1 < NROUND:
            issue_round(c + 1, (c + 1) % 2)
        drain_round(c, c % 2)
        compute_round(c, c % 2)

    pltpu.sync_copy(outv, out_hbm.at[pl.ds(base, BPW)])


@jax.jit
def _dist_mult_sc(idx3, ent3, rel3):
    mesh = plsc.VectorSubcoreMesh(core_axis_name="c", subcore_axis_name="s")
    kfn = functools.partial(
        pl.kernel,
        out_type=jax.ShapeDtypeStruct((BATCH,), jnp.float32),
        mesh=mesh,
        scratch_types=[
            pltpu.VMEM((1, 3, 1, BPW), jnp.int32),
            pltpu.VMEM((2, ROUND, DIM), jnp.float32),
            pltpu.VMEM((2, ROUND, DIM), jnp.float32),
            pltpu.VMEM((2, ROUND, DIM), jnp.float32),
            pltpu.VMEM((BPW,), jnp.float32),
            pltpu.SemaphoreType.DMA,
            pltpu.SemaphoreType.DMA,
        ],
    )(_dist_mult_body)
    return kfn(idx3, ent3, rel3)


def kernel(h, r, t, ent_emb, rel_emb):
    hrt = jnp.stack([jnp.asarray(h, jnp.int32),
                     jnp.asarray(r, jnp.int32),
                     jnp.asarray(t, jnp.int32)], axis=0)
    idx3 = hrt.reshape(3, NW, BPW).transpose(1, 0, 2).reshape(NW, 3, 1, BPW)
    ent3 = ent_emb.reshape(ent_emb.shape[0] // 8, 8, DIM)
    rel3 = rel_emb.reshape(rel_emb.shape[0] // 8, 8, DIM)
    return _dist_mult_sc(idx3, ent3, rel3)
